# baseline (speedup 1.0000x reference)
import functools

import jax
import jax.numpy as jnp
import numpy as np
from jax import lax
from jax.experimental import pallas as pl
from jax.experimental.pallas import tpu as pltpu
from jax.experimental.pallas import tpu_sc as plsc

VOCAB = 1000000
D = 64
BATCH = 4096
SEQ = 200
B_FLAT = BATCH * SEQ

NUM_WORKERS = 32
ROWS_PER_W = B_FLAT // NUM_WORKERS
CHUNK = 400
N_CHUNKS = ROWS_PER_W // CHUNK
GSLICE = 80
N_GS = CHUNK // GSLICE
LANES = 16


def _sinusoidal_pe_np(max_len, d_model):
    pos = np.arange(max_len, dtype=np.float32)[:, None]
    div = np.exp(np.arange(0, d_model, 2, dtype=np.float32) * (-np.log(10000.0) / d_model))
    pe = np.zeros((max_len, d_model), dtype=np.float32)
    pe[:, 0::2] = np.sin(pos * div)
    pe[:, 1::2] = np.cos(pos * div)
    return pe


_PE = _sinusoidal_pe_np(SEQ, D)


def _emb_body(table_hbm, idx_hbm, pe_hbm, out_hbm, idx_v, rows_v, pe_v, sem):
    wid = lax.axis_index("s") * 2 + lax.axis_index("c")
    base = wid * ROWS_PER_W

    pltpu.sync_copy(pe_hbm, pe_v)

    def chunk_body(c, _):
        row0 = base + c * CHUNK
        pltpu.sync_copy(idx_hbm.at[pl.ds(row0, CHUNK)], idx_v)
        copies = []
        for s in range(N_GS):
            copies.append(
                pltpu.async_copy(
                    table_hbm.at[idx_v.at[pl.ds(s * GSLICE, GSLICE)]],
                    rows_v.at[pl.ds(s * GSLICE, GSLICE)],
                    sem,
                )
            )
        for cp in copies:
            cp.wait()

        def pe_row(r, _):
            for col in range(D // LANES):
                pvec = pe_v[r, pl.ds(col * LANES, LANES)]
                for rep in range(CHUNK // SEQ):
                    plsc.addupdate(
                        rows_v.at[rep * SEQ + r, pl.ds(col * LANES, LANES)], pvec
                    )
            return 0

        lax.fori_loop(0, SEQ, pe_row, 0)

        pltpu.sync_copy(rows_v, out_hbm.at[pl.ds(row0, CHUNK)])
        return 0

    lax.fori_loop(0, N_CHUNKS, chunk_body, 0)


_mesh = plsc.VectorSubcoreMesh(core_axis_name="c", subcore_axis_name="s")

_emb = functools.partial(
    pl.kernel,
    mesh=_mesh,
    out_type=jax.ShapeDtypeStruct((B_FLAT, D), jnp.float32),
    compiler_params=pltpu.CompilerParams(use_tc_tiling_on_sc=False),
    scratch_types=[
        pltpu.VMEM((CHUNK,), jnp.int32),
        pltpu.VMEM((CHUNK, D), jnp.float32),
        pltpu.VMEM((SEQ, D), jnp.float32),
        pltpu.SemaphoreType.DMA,
    ],
)(_emb_body)


def kernel(input, table):
    idx = input.reshape(B_FLAT).astype(jnp.int32)
    out = _emb(table, idx, jnp.asarray(_PE))
    return out.reshape(BATCH, SEQ, D)


# speedup vs baseline: 1.0796x; 1.0796x over previous
import functools

import jax
import jax.numpy as jnp
import numpy as np
from jax import lax
from jax.experimental import pallas as pl
from jax.experimental.pallas import tpu as pltpu
from jax.experimental.pallas import tpu_sc as plsc

VOCAB = 1000000
D = 64
BATCH = 4096
SEQ = 200
B_FLAT = BATCH * SEQ

NUM_WORKERS = 32
ROWS_PER_W = B_FLAT // NUM_WORKERS
CHUNK = 400
N_CHUNKS = ROWS_PER_W // CHUNK
GSLICE = 80
N_GS = CHUNK // GSLICE
LANES = 16


def _sinusoidal_pe_np(max_len, d_model):
    pos = np.arange(max_len, dtype=np.float32)[:, None]
    div = np.exp(np.arange(0, d_model, 2, dtype=np.float32) * (-np.log(10000.0) / d_model))
    pe = np.zeros((max_len, d_model), dtype=np.float32)
    pe[:, 0::2] = np.sin(pos * div)
    pe[:, 1::2] = np.cos(pos * div)
    return pe


_PE = _sinusoidal_pe_np(SEQ, D)


def _emb_body(table_hbm, idx_hbm, pe_hbm, out_hbm, idx_v, rows_v, pe_v, sem0, sem1):
    wid = lax.axis_index("s") * 2 + lax.axis_index("c")
    base = wid * ROWS_PER_W
    sems = (sem0, sem1)

    pltpu.sync_copy(pe_hbm, pe_v)

    def fire(buf, row0):
        pltpu.sync_copy(idx_hbm.at[pl.ds(row0, CHUNK)], idx_v.at[buf])
        for s in range(N_GS):
            pltpu.async_copy(
                table_hbm.at[idx_v.at[buf].at[pl.ds(s * GSLICE, GSLICE)]],
                rows_v.at[buf].at[pl.ds(s * GSLICE, GSLICE)],
                sems[buf],
            )

    def drain(buf):
        for s in range(N_GS):
            pltpu.make_async_copy(
                table_hbm.at[idx_v.at[buf].at[pl.ds(s * GSLICE, GSLICE)]],
                rows_v.at[buf].at[pl.ds(s * GSLICE, GSLICE)],
                sems[buf],
            ).wait()

    def finish(buf, row0):
        def pe_row(r, _):
            for col in range(D // LANES):
                pvec = pe_v[r, pl.ds(col * LANES, LANES)]
                for rep in range(CHUNK // SEQ):
                    plsc.addupdate(
                        rows_v.at[buf, rep * SEQ + r, pl.ds(col * LANES, LANES)],
                        pvec,
                    )
            return 0

        lax.fori_loop(0, SEQ, pe_row, 0)
        pltpu.sync_copy(rows_v.at[buf], out_hbm.at[pl.ds(row0, CHUNK)])

    fire(0, base)

    def pair(i, _):
        a = base + (2 * i) * CHUNK
        fire(1, a + CHUNK)
        drain(0)
        finish(0, a)
        fire(0, a + 2 * CHUNK)
        drain(1)
        finish(1, a + CHUNK)
        return 0

    lax.fori_loop(0, N_CHUNKS // 2 - 1, pair, 0)
    a = base + (N_CHUNKS - 2) * CHUNK
    fire(1, a + CHUNK)
    drain(0)
    finish(0, a)
    drain(1)
    finish(1, a + CHUNK)


_mesh = plsc.VectorSubcoreMesh(core_axis_name="c", subcore_axis_name="s")

_emb = functools.partial(
    pl.kernel,
    mesh=_mesh,
    out_type=jax.ShapeDtypeStruct((B_FLAT, D), jnp.float32),
    compiler_params=pltpu.CompilerParams(use_tc_tiling_on_sc=False),
    scratch_types=[
        pltpu.VMEM((2, CHUNK), jnp.int32),
        pltpu.VMEM((2, CHUNK, D), jnp.float32),
        pltpu.VMEM((SEQ, D), jnp.float32),
        pltpu.SemaphoreType.DMA,
        pltpu.SemaphoreType.DMA,
    ],
)(_emb_body)


def kernel(input, table):
    idx = input.reshape(B_FLAT).astype(jnp.int32)
    out = _emb(table, idx, jnp.asarray(_PE))
    return out.reshape(BATCH, SEQ, D)


# speedup vs baseline: 1.4182x; 1.3137x over previous
import functools

import jax
import jax.numpy as jnp
import numpy as np
from jax import lax
from jax.experimental import pallas as pl
from jax.experimental.pallas import tpu as pltpu
from jax.experimental.pallas import tpu_sc as plsc

VOCAB = 1000000
D = 64
DP = 128
BATCH = 4096
SEQ = 200
B_FLAT = BATCH * SEQ

NUM_WORKERS = 32
ROWS_PER_W = B_FLAT // NUM_WORKERS
CHUNK = 400
N_CHUNKS = ROWS_PER_W // CHUNK
GSLICE = 80
N_GS = CHUNK // GSLICE
LANES = 16


def _sinusoidal_pe_np(max_len, d_model):
    pos = np.arange(max_len, dtype=np.float32)[:, None]
    div = np.exp(np.arange(0, d_model, 2, dtype=np.float32) * (-np.log(10000.0) / d_model))
    pe = np.zeros((max_len, d_model), dtype=np.float32)
    pe[:, 0::2] = np.sin(pos * div)
    pe[:, 1::2] = np.cos(pos * div)
    return pe


_PE = _sinusoidal_pe_np(SEQ, D)


def _emb_body(table_hbm, idx_hbm, pe_hbm, out_hbm, idx_v, rows_v, pe_v, sem0, sem1):
    wid = lax.axis_index("s") * 2 + lax.axis_index("c")
    base = wid * ROWS_PER_W
    sems = (sem0, sem1)

    pltpu.sync_copy(pe_hbm, pe_v)

    def fire(buf, row0):
        pltpu.sync_copy(idx_hbm.at[pl.ds(row0, CHUNK)], idx_v.at[buf])
        for s in range(N_GS):
            pltpu.async_copy(
                table_hbm.at[idx_v.at[buf].at[pl.ds(s * GSLICE, GSLICE)]],
                rows_v.at[buf].at[pl.ds(s * GSLICE, GSLICE)],
                sems[buf],
            )

    def drain(buf):
        for s in range(N_GS):
            pltpu.make_async_copy(
                table_hbm.at[idx_v.at[buf].at[pl.ds(s * GSLICE, GSLICE)]],
                rows_v.at[buf].at[pl.ds(s * GSLICE, GSLICE)],
                sems[buf],
            ).wait()

    def finish(buf, row0):
        def pe_row(r, _):
            for col in range(D // LANES):
                pvec = pe_v[r, pl.ds(col * LANES, LANES)]
                for rep in range(CHUNK // SEQ):
                    plsc.addupdate(
                        rows_v.at[buf, rep * SEQ + r, pl.ds(col * LANES, LANES)],
                        pvec,
                    )
            return 0

        lax.fori_loop(0, SEQ, pe_row, 0)
        pltpu.sync_copy(
            rows_v.at[buf],
            out_hbm.at[pl.ds(row0, CHUNK)].at[:, pl.ds(0, D)],
        )

    fire(0, base)

    def pair(i, _):
        a = base + (2 * i) * CHUNK
        fire(1, a + CHUNK)
        drain(0)
        finish(0, a)
        fire(0, a + 2 * CHUNK)
        drain(1)
        finish(1, a + CHUNK)
        return 0

    lax.fori_loop(0, N_CHUNKS // 2 - 1, pair, 0)
    a = base + (N_CHUNKS - 2) * CHUNK
    fire(1, a + CHUNK)
    drain(0)
    finish(0, a)
    drain(1)
    finish(1, a + CHUNK)


_mesh = plsc.VectorSubcoreMesh(core_axis_name="c", subcore_axis_name="s")

_emb = functools.partial(
    pl.kernel,
    mesh=_mesh,
    out_type=jax.ShapeDtypeStruct((B_FLAT, DP), jnp.float32),
    compiler_params=pltpu.CompilerParams(use_tc_tiling_on_sc=False),
    scratch_types=[
        pltpu.VMEM((2, CHUNK), jnp.int32),
        pltpu.VMEM((2, CHUNK, D), jnp.float32),
        pltpu.VMEM((SEQ, D), jnp.float32),
        pltpu.SemaphoreType.DMA,
        pltpu.SemaphoreType.DMA,
    ],
)(_emb_body)


def kernel(input, table):
    idx = input.reshape(B_FLAT).astype(jnp.int32)
    out = _emb(table, idx, jnp.asarray(_PE))
    return out[:, :D].reshape(BATCH, SEQ, D)


# speedup vs baseline: 1.4854x; 1.0474x over previous
import functools

import jax
import jax.numpy as jnp
import numpy as np
from jax import lax
from jax.experimental import pallas as pl
from jax.experimental.pallas import tpu as pltpu
from jax.experimental.pallas import tpu_sc as plsc

VOCAB = 1000000
D = 64
DP = 128
BATCH = 4096
SEQ = 200
B_FLAT = BATCH * SEQ

NUM_WORKERS = 32
ROWS_PER_W = B_FLAT // NUM_WORKERS
CHUNK = 400
N_CHUNKS = ROWS_PER_W // CHUNK
GSLICE = 80
N_GS = CHUNK // GSLICE
LANES = 16


def _sinusoidal_pe_np(max_len, d_model):
    pos = np.arange(max_len, dtype=np.float32)[:, None]
    div = np.exp(np.arange(0, d_model, 2, dtype=np.float32) * (-np.log(10000.0) / d_model))
    pe = np.zeros((max_len, d_model), dtype=np.float32)
    pe[:, 0::2] = np.sin(pos * div)
    pe[:, 1::2] = np.cos(pos * div)
    return pe


_PE = _sinusoidal_pe_np(SEQ, D)


NBUF = 3


def _emb_body(table_hbm, idx_hbm, pe_hbm, out_hbm, idx_v, rows_v, pe_v,
              g0, g1, g2, w0, w1, w2):
    wid = lax.axis_index("s") * 2 + lax.axis_index("c")
    base = wid * ROWS_PER_W
    gsems = (g0, g1, g2)
    wsems = (w0, w1, w2)

    pltpu.sync_copy(pe_hbm, pe_v)

    def fire(buf, row0):
        pltpu.sync_copy(idx_hbm.at[pl.ds(row0, CHUNK)], idx_v.at[buf])
        for s in range(N_GS):
            pltpu.async_copy(
                table_hbm.at[idx_v.at[buf].at[pl.ds(s * GSLICE, GSLICE)]],
                rows_v.at[buf].at[pl.ds(s * GSLICE, GSLICE)],
                gsems[buf],
            )

    def drain(buf):
        for s in range(N_GS):
            pltpu.make_async_copy(
                table_hbm.at[idx_v.at[buf].at[pl.ds(s * GSLICE, GSLICE)]],
                rows_v.at[buf].at[pl.ds(s * GSLICE, GSLICE)],
                gsems[buf],
            ).wait()

    def pe_add(buf):
        def pe_row(r, _):
            for col in range(D // LANES):
                pvec = pe_v[r, pl.ds(col * LANES, LANES)]
                for rep in range(CHUNK // SEQ):
                    plsc.addupdate(
                        rows_v.at[buf, rep * SEQ + r, pl.ds(col * LANES, LANES)],
                        pvec,
                    )
            return 0

        lax.fori_loop(0, SEQ, pe_row, 0)

    def wfire(buf, row0):
        pltpu.async_copy(
            rows_v.at[buf],
            out_hbm.at[pl.ds(row0, CHUNK)].at[:, pl.ds(0, D)],
            wsems[buf],
        )

    def wwait(buf, row0):
        pltpu.make_async_copy(
            rows_v.at[buf],
            out_hbm.at[pl.ds(row0, CHUNK)].at[:, pl.ds(0, D)],
            wsems[buf],
        ).wait()

    def row_of(c):
        return base + c * CHUNK

    def step(c, waitw, firenext):
        b = c % NBUF
        drain(b)
        pe_add(b)
        wfire(b, row_of(c))
        if firenext:
            bn = (c + 2) % NBUF
            if waitw:
                wwait(bn, row_of(c - 1))
            fire(bn, row_of(c + 2))

    fire(0, base)
    fire(1, base + CHUNK)
    step(0, False, True)
    step(1, True, True)
    step(2, True, True)

    def group(g, _):
        c0 = 3 * g

        def gstep(k):
            c = c0 + k
            b = k
            drain(b)
            pe_add(b)
            wfire(b, row_of(c))
            bn = (k + 2) % NBUF
            wwait(bn, row_of(c - 1))
            fire(bn, row_of(c + 2))

        gstep(0)
        gstep(1)
        gstep(2)
        return 0

    lax.fori_loop(1, 20, group, 0)

    step(60, True, True)
    step(61, True, True)
    step(62, False, False)
    step(63, False, False)
    wwait(61 % NBUF, row_of(61))
    wwait(62 % NBUF, row_of(62))
    wwait(63 % NBUF, row_of(63))


_mesh = plsc.VectorSubcoreMesh(core_axis_name="c", subcore_axis_name="s")

_emb = functools.partial(
    pl.kernel,
    mesh=_mesh,
    out_type=jax.ShapeDtypeStruct((B_FLAT, DP), jnp.float32),
    compiler_params=pltpu.CompilerParams(use_tc_tiling_on_sc=False),
    scratch_types=[
        pltpu.VMEM((NBUF, CHUNK), jnp.int32),
        pltpu.VMEM((NBUF, CHUNK, D), jnp.float32),
        pltpu.VMEM((SEQ, D), jnp.float32),
        pltpu.SemaphoreType.DMA,
        pltpu.SemaphoreType.DMA,
        pltpu.SemaphoreType.DMA,
        pltpu.SemaphoreType.DMA,
        pltpu.SemaphoreType.DMA,
        pltpu.SemaphoreType.DMA,
    ],
)(_emb_body)


def kernel(input, table):
    idx = input.reshape(B_FLAT).astype(jnp.int32)
    out = _emb(table, idx, jnp.asarray(_PE))
    return out[:, :D].reshape(BATCH, SEQ, D)


# speedup vs baseline: 1.4926x; 1.0048x over previous
import functools

import jax
import jax.numpy as jnp
import numpy as np
from jax import lax
from jax.experimental import pallas as pl
from jax.experimental.pallas import tpu as pltpu
from jax.experimental.pallas import tpu_sc as plsc

VOCAB = 1000000
D = 64
DP = 128
BATCH = 4096
SEQ = 200
B_FLAT = BATCH * SEQ

NUM_WORKERS = 32
ROWS_PER_W = B_FLAT // NUM_WORKERS
CHUNK = 400
N_CHUNKS = ROWS_PER_W // CHUNK
GSLICE = 80
N_GS = CHUNK // GSLICE
LANES = 16


def _sinusoidal_pe_np(max_len, d_model):
    pos = np.arange(max_len, dtype=np.float32)[:, None]
    div = np.exp(np.arange(0, d_model, 2, dtype=np.float32) * (-np.log(10000.0) / d_model))
    pe = np.zeros((max_len, d_model), dtype=np.float32)
    pe[:, 0::2] = np.sin(pos * div)
    pe[:, 1::2] = np.cos(pos * div)
    return pe


_PE = _sinusoidal_pe_np(SEQ, D)


NBUF = 3


def _emb_body(table_hbm, idx_hbm, pe_hbm, out_hbm, idx_v, rows_v, pe_v,
              g0, g1, g2, w0, w1, w2):
    wid = lax.axis_index("s") * 2 + lax.axis_index("c")
    base = wid * ROWS_PER_W
    gsems = (g0, g1, g2)
    wsems = (w0, w1, w2)

    pltpu.sync_copy(pe_hbm, pe_v)
    pltpu.sync_copy(idx_hbm.at[pl.ds(base, ROWS_PER_W)], idx_v)

    def fire(buf, row0):
        off = row0 - base
        for s in range(N_GS):
            pltpu.async_copy(
                table_hbm.at[idx_v.at[pl.ds(off + s * GSLICE, GSLICE)]],
                rows_v.at[buf].at[pl.ds(s * GSLICE, GSLICE)],
                gsems[buf],
            )

    def drain(buf, row0):
        off = row0 - base
        for s in range(N_GS):
            pltpu.make_async_copy(
                table_hbm.at[idx_v.at[pl.ds(off + s * GSLICE, GSLICE)]],
                rows_v.at[buf].at[pl.ds(s * GSLICE, GSLICE)],
                gsems[buf],
            ).wait()

    def pe_add(buf):
        def pe_row(r, _):
            for col in range(D // LANES):
                pvec = pe_v[r, pl.ds(col * LANES, LANES)]
                for rep in range(CHUNK // SEQ):
                    plsc.addupdate(
                        rows_v.at[buf, rep * SEQ + r, pl.ds(col * LANES, LANES)],
                        pvec,
                    )
            return 0

        lax.fori_loop(0, SEQ, pe_row, 0)

    def wfire(buf, row0):
        pltpu.async_copy(
            rows_v.at[buf],
            out_hbm.at[pl.ds(row0, CHUNK)].at[:, pl.ds(0, D)],
            wsems[buf],
        )

    def wwait(buf, row0):
        pltpu.make_async_copy(
            rows_v.at[buf],
            out_hbm.at[pl.ds(row0, CHUNK)].at[:, pl.ds(0, D)],
            wsems[buf],
        ).wait()

    def row_of(c):
        return base + c * CHUNK

    def step(c, waitw, firenext):
        b = c % NBUF
        drain(b, row_of(c))
        pe_add(b)
        wfire(b, row_of(c))
        if firenext:
            bn = (c + 2) % NBUF
            if waitw:
                wwait(bn, row_of(c - 1))
            fire(bn, row_of(c + 2))

    fire(0, base)
    fire(1, base + CHUNK)
    step(0, False, True)
    step(1, True, True)
    step(2, True, True)

    def group(g, _):
        c0 = 3 * g

        def gstep(k):
            c = c0 + k
            b = k
            drain(b, row_of(c))
            pe_add(b)
            wfire(b, row_of(c))
            bn = (k + 2) % NBUF
            wwait(bn, row_of(c - 1))
            fire(bn, row_of(c + 2))

        gstep(0)
        gstep(1)
        gstep(2)
        return 0

    lax.fori_loop(1, 20, group, 0)

    step(60, True, True)
    step(61, True, True)
    step(62, False, False)
    step(63, False, False)
    wwait(61 % NBUF, row_of(61))
    wwait(62 % NBUF, row_of(62))
    wwait(63 % NBUF, row_of(63))


_mesh = plsc.VectorSubcoreMesh(core_axis_name="c", subcore_axis_name="s")

_emb = functools.partial(
    pl.kernel,
    mesh=_mesh,
    out_type=jax.ShapeDtypeStruct((B_FLAT, DP), jnp.float32),
    compiler_params=pltpu.CompilerParams(use_tc_tiling_on_sc=False),
    scratch_types=[
        pltpu.VMEM((ROWS_PER_W,), jnp.int32),
        pltpu.VMEM((NBUF, CHUNK, D), jnp.float32),
        pltpu.VMEM((SEQ, D), jnp.float32),
        pltpu.SemaphoreType.DMA,
        pltpu.SemaphoreType.DMA,
        pltpu.SemaphoreType.DMA,
        pltpu.SemaphoreType.DMA,
        pltpu.SemaphoreType.DMA,
        pltpu.SemaphoreType.DMA,
    ],
)(_emb_body)


def kernel(input, table):
    idx = input.reshape(B_FLAT).astype(jnp.int32)
    out = _emb(table, idx, jnp.asarray(_PE))
    return out[:, :D].reshape(BATCH, SEQ, D)
